# Initial kernel scaffold; baseline (speedup 1.0000x reference)
import functools

import jax
import jax.numpy as jnp
from jax import lax
from jax.experimental import pallas as pl
from jax.experimental.pallas import tpu as pltpu
from jax.experimental.pallas import tpu_sc as plsc

N = 10000
E = 320000
CHUNK = 128
NC, NS = 2, 16
NW = NC * NS
CPW = -(-E // (NW * CHUNK))
NCHUNK = NW * CPW
E_PAD = NCHUNK * CHUNK
ACC_ROWS = 10240
ROWS_PER_TILE_OUT = N // NS

_mesh = plsc.VectorSubcoreMesh(core_axis_name="c", subcore_axis_name="s")


def _deg_body(dst_hbm, out_hbm, idxbuf, onesbuf, acc):
    c = lax.axis_index("c")
    s = lax.axis_index("s")
    lane = lax.iota(jnp.int32, 16)
    onerow = jnp.where(lane == 0, 1.0, 0.0)
    zrow = jnp.zeros((16,), jnp.float32)

    def fill(i, _):
        onesbuf[i, :] = onerow
        return 0

    lax.fori_loop(0, CHUNK, fill, 0)

    def zacc(i, _):
        acc[s * 640 + i, :] = zrow
        return 0

    lax.fori_loop(0, 640, zacc, 0)
    plsc.subcore_barrier()

    base = (c * NS + s) * CPW

    def step(i, _):
        pltpu.sync_copy(dst_hbm.at[base + i], idxbuf)
        pltpu.sync_copy(onesbuf, acc.at[idxbuf], add=True)
        return 0

    lax.fori_loop(0, CPW, step, 0)
    plsc.subcore_barrier()
    r0 = s * ROWS_PER_TILE_OUT
    pltpu.sync_copy(acc.at[pl.ds(r0, ROWS_PER_TILE_OUT)],
                    out_hbm.at[c, pl.ds(r0, ROWS_PER_TILE_OUT)])


_deg_kernel = functools.partial(
    pl.kernel,
    out_type=jax.ShapeDtypeStruct((NC, N, 16), jnp.float32),
    mesh=_mesh,
    scratch_types=[
        pltpu.VMEM((CHUNK,), jnp.int32),
        pltpu.VMEM((CHUNK, 16), jnp.float32),
        pltpu.VMEM_SHARED((ACC_ROWS, 16), jnp.float32),
    ],
)(_deg_body)


def _agg_body(src_hbm, dst_hbm, g_hbm, out_hbm, sidx, didx, rows, acc, sem):
    c = lax.axis_index("c")
    s = lax.axis_index("s")
    zrow = jnp.zeros((16,), jnp.float32)

    def zfill(i, _):
        for j in range(8):
            rows[i, pl.ds(j * 16, 16)] = zrow
        return 0

    lax.fori_loop(0, CHUNK, zfill, 0)
    for k in range(640 // CHUNK):
        pltpu.sync_copy(rows, acc.at[pl.ds(s * 640 + k * CHUNK, CHUNK)])
    plsc.subcore_barrier()

    base = (c * NS + s) * CPW

    def step(i, _):
        cid = base + i
        pltpu.sync_copy(src_hbm.at[cid], sidx)
        pltpu.sync_copy(dst_hbm.at[cid], didx)
        pltpu.async_copy(g_hbm.at[sidx], rows, sem).wait()
        pltpu.sync_copy(rows, acc.at[didx], add=True)
        return 0

    lax.fori_loop(0, CPW, step, 0)
    plsc.subcore_barrier()
    r0 = s * ROWS_PER_TILE_OUT
    pltpu.sync_copy(acc.at[pl.ds(r0, ROWS_PER_TILE_OUT)],
                    out_hbm.at[c, pl.ds(r0, ROWS_PER_TILE_OUT)])


_agg_kernel = functools.partial(
    pl.kernel,
    out_type=jax.ShapeDtypeStruct((NC, N, 128), jnp.float32),
    mesh=_mesh,
    scratch_types=[
        pltpu.VMEM((CHUNK,), jnp.int32),
        pltpu.VMEM((CHUNK,), jnp.int32),
        pltpu.VMEM((CHUNK, 128), jnp.float32),
        pltpu.VMEM_SHARED((ACC_ROWS, 128), jnp.float32),
        pltpu.SemaphoreType.DMA,
    ],
)(_agg_body)


_BR = 1000


def _dis_block(d0r, d1r):
    deg = d0r[:, 0:1] + d1r[:, 0:1] + 1.0
    return lax.rsqrt(deg)


def _scale_mm_body(xr, wr, d0r, d1r, gr):
    h = jnp.dot(xr[...], wr[...], preferred_element_type=jnp.float32)
    gr[...] = h * _dis_block(d0r, d1r)


def _layer2_body(a0r, a1r, gr, d0r, d1r, br, wr, or_):
    dis = _dis_block(d0r, d1r)
    h = jnp.tanh(dis * (a0r[...] + a1r[...] + gr[...]) + br[...])
    or_[...] = jnp.dot(h, wr[...], preferred_element_type=jnp.float32) * dis


def _final_body(a0r, a1r, gr, d0r, d1r, br, wcr, bcr, outr, hr):
    dis = _dis_block(d0r, d1r)
    h = jnp.tanh(dis * (a0r[...] + a1r[...] + gr[...]) + br[...])
    hr[...] = h
    outr[...] = jnp.dot(h, wcr[...], preferred_element_type=jnp.float32) + bcr[...]


def _rows(bs):
    return pl.BlockSpec((_BR, bs), lambda i: (i, 0))


def _full(shape):
    return pl.BlockSpec(shape, lambda i: tuple(0 for _ in shape))


def kernel(x, edge_index, W1, b1, W2, b2, Wc, bc):
    src = edge_index[0]
    dst = edge_index[1]
    pad = E_PAD - E
    src_p = jnp.concatenate([src, jnp.zeros((pad,), jnp.int32)]).reshape(NCHUNK, CHUNK)
    dst_p = jnp.concatenate([dst, jnp.full((pad,), N, jnp.int32)]).reshape(NCHUNK, CHUNK)
    b1r = b1.reshape(1, 128)
    b2r = b2.reshape(1, 128)
    bcr = bc.reshape(1, 64)

    deg_parts = _deg_kernel(dst_p)
    d0, d1 = deg_parts[0], deg_parts[1]

    grid = N // _BR
    g1 = pl.pallas_call(
        _scale_mm_body,
        grid=(grid,),
        in_specs=[_rows(128), _full((128, 128)), _rows(16), _rows(16)],
        out_specs=_rows(128),
        out_shape=jax.ShapeDtypeStruct((N, 128), jnp.float32),
    )(x, W1, d0, d1)

    agg1 = _agg_kernel(src_p, dst_p, g1)

    g2 = pl.pallas_call(
        _layer2_body,
        grid=(grid,),
        in_specs=[_rows(128), _rows(128), _rows(128), _rows(16), _rows(16),
                  _full((1, 128)), _full((128, 128))],
        out_specs=_rows(128),
        out_shape=jax.ShapeDtypeStruct((N, 128), jnp.float32),
    )(agg1[0], agg1[1], g1, d0, d1, b1r, W2)

    agg2 = _agg_kernel(src_p, dst_p, g2)

    out, h2 = pl.pallas_call(
        _final_body,
        grid=(grid,),
        in_specs=[_rows(128), _rows(128), _rows(128), _rows(16), _rows(16),
                  _full((1, 128)), _full((128, 64)), _full((1, 64))],
        out_specs=[_rows(64), _rows(128)],
        out_shape=[jax.ShapeDtypeStruct((N, 64), jnp.float32),
                   jax.ShapeDtypeStruct((N, 128), jnp.float32)],
    )(agg2[0], agg2[1], g2, d0, d1, b2r, Wc, bcr)

    return (out, h2)


# speedup vs baseline: 10.4209x; 10.4209x over previous
import functools

import jax
import jax.numpy as jnp
from jax import lax
from jax.experimental import pallas as pl
from jax.experimental.pallas import tpu as pltpu
from jax.experimental.pallas import tpu_sc as plsc

N = 10000
E = 320000
CHUNK = 128
NC, NS = 2, 16
NW = NC * NS
CPW = -(-E // (NW * CHUNK))
NCHUNK = NW * CPW
E_PAD = NCHUNK * CHUNK
ACC_ROWS = 10240
RPT = ACC_ROWS // NS

_mesh = plsc.VectorSubcoreMesh(core_axis_name="c", subcore_axis_name="s")


def _deg_body(dst_hbm, out_hbm, didx, rows, acc):
    c = lax.axis_index("c")
    s = lax.axis_index("s")
    zrow = jnp.zeros((16,), jnp.float32)
    onerow = jnp.full((16,), 1.0, jnp.float32)

    def zfill(i, _):
        for j in range(8):
            rows[i, pl.ds(j * 16, 16)] = zrow
        return 0

    lax.fori_loop(0, CHUNK, zfill, 0)
    for k in range(RPT // CHUNK):
        pltpu.sync_copy(rows, acc.at[pl.ds(s * RPT + k * CHUNK, CHUNK)])

    def ofill(i, _):
        for j in range(8):
            rows[i, pl.ds(j * 16, 16)] = onerow
        return 0

    lax.fori_loop(0, CHUNK, ofill, 0)
    plsc.subcore_barrier()

    base = (c * NS + s) * CPW

    def step(i, _):
        pltpu.sync_copy(dst_hbm.at[base + i], didx)
        pltpu.sync_copy(rows, acc.at[didx], add=True)
        return 0

    lax.fori_loop(0, CPW, step, 0)
    plsc.subcore_barrier()
    r0 = s * RPT
    pltpu.sync_copy(acc.at[pl.ds(r0, RPT)], out_hbm.at[c, pl.ds(r0, RPT)])


_deg_kernel = functools.partial(
    pl.kernel,
    out_type=jax.ShapeDtypeStruct((NC, ACC_ROWS, 128), jnp.float32),
    mesh=_mesh,
    scratch_types=[
        pltpu.VMEM((CHUNK,), jnp.int32),
        pltpu.VMEM((CHUNK, 128), jnp.float32),
        pltpu.VMEM_SHARED((ACC_ROWS, 128), jnp.float32),
    ],
)(_deg_body)


def _agg_body(src_hbm, dst_hbm, g_hbm, out_hbm, sidx, didx, rows, acc, sem):
    c = lax.axis_index("c")
    s = lax.axis_index("s")
    zrow = jnp.zeros((16,), jnp.float32)

    def zfill(i, _):
        for j in range(8):
            rows[i, pl.ds(j * 16, 16)] = zrow
        return 0

    lax.fori_loop(0, CHUNK, zfill, 0)
    for k in range(640 // CHUNK):
        pltpu.sync_copy(rows, acc.at[pl.ds(s * 640 + k * CHUNK, CHUNK)])
    plsc.subcore_barrier()

    base = (c * NS + s) * CPW

    def step(i, _):
        cid = base + i
        pltpu.sync_copy(src_hbm.at[cid], sidx)
        pltpu.sync_copy(dst_hbm.at[cid], didx)
        pltpu.async_copy(g_hbm.at[sidx], rows, sem).wait()
        pltpu.sync_copy(rows, acc.at[didx], add=True)
        return 0

    lax.fori_loop(0, CPW, step, 0)
    plsc.subcore_barrier()
    r0 = s * RPT
    pltpu.sync_copy(acc.at[pl.ds(r0, RPT)], out_hbm.at[c, pl.ds(r0, RPT)])


_agg_kernel = functools.partial(
    pl.kernel,
    out_type=jax.ShapeDtypeStruct((NC, ACC_ROWS, 128), jnp.float32),
    mesh=_mesh,
    scratch_types=[
        pltpu.VMEM((CHUNK,), jnp.int32),
        pltpu.VMEM((CHUNK,), jnp.int32),
        pltpu.VMEM((CHUNK, 128), jnp.float32),
        pltpu.VMEM_SHARED((ACC_ROWS, 128), jnp.float32),
        pltpu.SemaphoreType.DMA,
    ],
)(_agg_body)


_BR = 1000


def _dis_block(d0r, d1r):
    deg = d0r[...] + d1r[...] + 1.0
    return lax.rsqrt(deg)


def _scale_mm_body(xr, wr, d0r, d1r, gr):
    h = jnp.dot(xr[...], wr[...], preferred_element_type=jnp.float32)
    gr[...] = h * _dis_block(d0r, d1r)


def _layer2_body(a0r, a1r, gr, d0r, d1r, br, wr, or_):
    dis = _dis_block(d0r, d1r)
    h = jnp.tanh(dis * (a0r[...] + a1r[...] + gr[...]) + br[...])
    or_[...] = jnp.dot(h, wr[...], preferred_element_type=jnp.float32) * dis


def _final_body(a0r, a1r, gr, d0r, d1r, br, wcr, bcr, outr, hr):
    dis = _dis_block(d0r, d1r)
    h = jnp.tanh(dis * (a0r[...] + a1r[...] + gr[...]) + br[...])
    hr[...] = h
    outr[...] = jnp.dot(h, wcr[...], preferred_element_type=jnp.float32) + bcr[...]


def _rows(bs):
    return pl.BlockSpec((_BR, bs), lambda i: (i, 0))


def _full(shape):
    return pl.BlockSpec(shape, lambda i: tuple(0 for _ in shape))


def kernel(x, edge_index, W1, b1, W2, b2, Wc, bc):
    src = edge_index[0]
    dst = edge_index[1]
    pad = E_PAD - E
    src_p = jnp.concatenate([src, jnp.zeros((pad,), jnp.int32)]).reshape(NCHUNK, CHUNK)
    dst_p = jnp.concatenate([dst, jnp.full((pad,), N, jnp.int32)]).reshape(NCHUNK, CHUNK)
    b1r = b1.reshape(1, 128)
    b2r = b2.reshape(1, 128)
    bcr = bc.reshape(1, 64)

    deg_parts = _deg_kernel(dst_p)
    d0, d1 = deg_parts[0], deg_parts[1]

    grid = N // _BR
    g1 = pl.pallas_call(
        _scale_mm_body,
        grid=(grid,),
        in_specs=[_rows(128), _full((128, 128)), _rows(128), _rows(128)],
        out_specs=_rows(128),
        out_shape=jax.ShapeDtypeStruct((N, 128), jnp.float32),
    )(x, W1, d0, d1)

    agg1 = _agg_kernel(src_p, dst_p, g1)

    g2 = pl.pallas_call(
        _layer2_body,
        grid=(grid,),
        in_specs=[_rows(128), _rows(128), _rows(128), _rows(128), _rows(128),
                  _full((1, 128)), _full((128, 128))],
        out_specs=_rows(128),
        out_shape=jax.ShapeDtypeStruct((N, 128), jnp.float32),
    )(agg1[0], agg1[1], g1, d0, d1, b1r, W2)

    agg2 = _agg_kernel(src_p, dst_p, g2)

    out, h2 = pl.pallas_call(
        _final_body,
        grid=(grid,),
        in_specs=[_rows(128), _rows(128), _rows(128), _rows(128), _rows(128),
                  _full((1, 128)), _full((128, 64)), _full((1, 64))],
        out_specs=[_rows(64), _rows(128)],
        out_shape=[jax.ShapeDtypeStruct((N, 64), jnp.float32),
                   jax.ShapeDtypeStruct((N, 128), jnp.float32)],
    )(agg2[0], agg2[1], g2, d0, d1, b2r, Wc, bcr)

    return (out, h2)
